# Initial kernel scaffold; baseline (speedup 1.0000x reference)
import jax
import jax.numpy as jnp
from jax.experimental import pallas as pl


def kernel(classifications, regressions, leftnesses, annotations):
    raise NotImplementedError("write your pallas kernel here")


# speedup vs baseline: 3.1627x; 3.1627x over previous
import jax
import jax.numpy as jnp
from jax import lax
from jax.experimental import pallas as pl

INF = 1e8
LEVEL_SIZES = (32768, 16384, 8192)
STRIDES = (1.0, 2.0, 4.0)
B = 2
M = 256
N_TOTAL = 57344
AUDIO_TARGET_RATE = 22050.0 / 256.0
BEAT_RADIUS = 2.5
DOWNBEAT_RADIUS = 4.5
EPS = 1e-6

_EDGE0 = 0.35 + (0.7 - 0.35) / 2.0
_EDGE1 = 0.7 + (1.4 - 0.7) / 2.0
RANGE_LO = (-1.0 * AUDIO_TARGET_RATE, _EDGE0 * AUDIO_TARGET_RATE, _EDGE1 * AUDIO_TARGET_RATE)
RANGE_HI = (_EDGE0 * AUDIO_TARGET_RATE, _EDGE1 * AUDIO_TARGET_RATE, 1000.0 * AUDIO_TARGET_RATE)

BLK = 1024
NBLK = N_TOTAL // BLK
_LVL0_BLKS = LEVEL_SIZES[0] // BLK
_LVL01_BLKS = (LEVEL_SIZES[0] + LEVEL_SIZES[1]) // BLK


def _loss_kernel(cls_ref, reg_ref, lef_ref, ann_ref, out_ref):
    j = pl.program_id(1)
    jf = j.astype(jnp.float32)

    is0 = j < _LVL0_BLKS
    is1 = (j >= _LVL0_BLKS) & (j < _LVL01_BLKS)
    s = jnp.where(is0, STRIDES[0], jnp.where(is1, STRIDES[1], STRIDES[2]))
    local0 = jnp.where(is0, jf * BLK,
                       jnp.where(is1, jf * BLK - LEVEL_SIZES[0],
                                 jf * BLK - (LEVEL_SIZES[0] + LEVEL_SIZES[1])))
    lo = jnp.where(is0, RANGE_LO[0], jnp.where(is1, RANGE_LO[1], RANGE_LO[2]))
    hi = jnp.where(is0, RANGE_HI[0], jnp.where(is1, RANGE_HI[1], RANGE_HI[2]))

    ann = ann_ref[0]
    l_ann = ann[0, :][None, :]
    r_ann = ann[1, :][None, :]
    c_ann = ann[2, :][None, :]
    radius = jnp.where(c_ann == 0.0, DOWNBEAT_RADIUS, BEAT_RADIUS)
    area = r_ann - l_ann

    ik = lax.broadcasted_iota(jnp.int32, (BLK, 1), 0).astype(jnp.float32)
    a = (local0 + ik) * s

    limit = l_ann + radius * s
    in_box = (a >= l_ann) & (a <= jnp.minimum(r_ann, limit))
    l_star = a - l_ann
    r_star = r_ann - a
    max_lr = jnp.maximum(l_star, r_star)
    valid = in_box & (max_lr >= lo) & (max_lr <= hi)
    area_m = jnp.where(valid, jnp.broadcast_to(area, valid.shape), INF)

    min_area = jnp.min(area_m, axis=1, keepdims=True)
    mask_eq = area_m == min_area
    m_iota = lax.broadcasted_iota(jnp.int32, (BLK, M), 1)
    idx = jnp.min(jnp.where(mask_eq, m_iota, M), axis=1, keepdims=True)
    onehot = (mask_eq & (m_iota == idx)).astype(jnp.float32)

    l_sel = jnp.sum(onehot * l_ann, axis=1)
    r_sel = jnp.sum(onehot * r_ann, axis=1)
    c_sel = jnp.sum(onehot * c_ann, axis=1)
    posf = (min_area[:, 0] < INF).astype(jnp.float32)
    av = a[:, 0]
    nl = (av - l_sel) / s
    nr = (r_sel - av) / s

    t1 = posf * (c_sel > 0.5).astype(jnp.float32)
    t0 = posf - t1
    p = jnp.clip(cls_ref[0], EPS, 1.0 - EPS)
    p0 = p[0]
    p1 = p[1]

    def _focal(t, q):
        pos_term = 0.25 * (1.0 - q) * (1.0 - q) * (-jnp.log(q))
        neg_term = 0.75 * q * q * (-jnp.log(1.0 - q))
        return jnp.where(t == 1.0, pos_term, neg_term)

    cls_sum = jnp.sum(_focal(t0, p0) + _focal(t1, p1))

    tl = jnp.maximum(nl, 1e-3)
    tr = jnp.maximum(nr, 1e-3)
    reg = reg_ref[0]
    pl_ = reg[0]
    pr_ = reg[1]
    inter = jnp.minimum(pl_, tl) + jnp.minimum(pr_, tr)
    union = jnp.maximum(pl_, tl) + jnp.maximum(pr_, tr)
    iou = jnp.clip(inter / (union + EPS), EPS, 1.0)
    reg_sum = jnp.sum(-jnp.log(iou) * posf)

    lt = jnp.clip(tr / (tl + tr + EPS), EPS, 1.0 - EPS)
    lp = jnp.clip(lef_ref[0, 0], EPS, 1.0 - EPS)
    lbce = -(lt * jnp.log(lp) + (1.0 - lt) * jnp.log(1.0 - lp))
    left_sum = jnp.sum(lbce * posf)

    pos_sum = jnp.sum(posf)

    lane = lax.broadcasted_iota(jnp.int32, (1, 1, 128), 2)
    acc = jnp.where(lane == 0, cls_sum,
                    jnp.where(lane == 1, reg_sum,
                              jnp.where(lane == 2, left_sum,
                                        jnp.where(lane == 3, pos_sum, 0.0))))

    @pl.when(j == 0)
    def _init():
        out_ref[...] = acc

    @pl.when(j > 0)
    def _acc():
        out_ref[...] += acc


def kernel(classifications, regressions, leftnesses, annotations):
    cls_t = classifications.transpose(0, 2, 1)
    reg_t = regressions.transpose(0, 2, 1)
    lef_t = leftnesses.transpose(0, 2, 1)
    ann_t = annotations.transpose(0, 2, 1)

    partials = pl.pallas_call(
        _loss_kernel,
        grid=(B, NBLK),
        in_specs=[
            pl.BlockSpec((1, 2, BLK), lambda b, j: (b, 0, j)),
            pl.BlockSpec((1, 2, BLK), lambda b, j: (b, 0, j)),
            pl.BlockSpec((1, 1, BLK), lambda b, j: (b, 0, j)),
            pl.BlockSpec((1, 3, M), lambda b, j: (b, 0, 0)),
        ],
        out_specs=pl.BlockSpec((1, 1, 128), lambda b, j: (b, 0, 0)),
        out_shape=jax.ShapeDtypeStruct((B, 1, 128), jnp.float32),
    )(cls_t, reg_t, lef_t, ann_t)

    sums = partials[:, 0, :4]
    num_pos = jnp.maximum(sums[:, 3], 1.0)
    per_batch = (sums[:, 0] + sums[:, 1] + sums[:, 2]) / num_pos
    return jnp.sum(per_batch) / float(B)


# speedup vs baseline: 20.8581x; 6.5950x over previous
import functools

import jax
import jax.numpy as jnp
from jax import lax
from jax.experimental import pallas as pl
from jax.experimental.pallas import tpu as pltpu
from jax.experimental.pallas import tpu_sc as plsc

INF = 1e8
LEVEL_SIZES = (32768, 16384, 8192)
LEVEL_OFFS = (0, 32768, 49152)
STRIDES = (1.0, 2.0, 4.0)
B = 2
M = 256
N_TOTAL = 57344
AUDIO_TARGET_RATE = 22050.0 / 256.0
BEAT_RADIUS = 2.5
DOWNBEAT_RADIUS = 4.5
EPS = 1e-6

_EDGE0 = 0.35 + (0.7 - 0.35) / 2.0
_EDGE1 = 0.7 + (1.4 - 0.7) / 2.0
RANGE_LO = (-1.0 * AUDIO_TARGET_RATE, _EDGE0 * AUDIO_TARGET_RATE, _EDGE1 * AUDIO_TARGET_RATE)
RANGE_HI = (_EDGE0 * AUDIO_TARGET_RATE, _EDGE1 * AUDIO_TARGET_RATE, 1000.0 * AUDIO_TARGET_RATE)

NW = 32
CMAX = LEVEL_SIZES[0] // NW
_PAD = 8


def _sc_assign_body(ann_hbm, out_hbm, ann_v, ba_v, bl_v, br_v, bc_v, st_v):
    wid = lax.axis_index("s") * 2 + lax.axis_index("c")
    pltpu.sync_copy(ann_hbm, ann_v)
    lane = lax.broadcasted_iota(jnp.int32, (16,), 0)

    for b in range(B):
        for lvl in range(3):
            C = LEVEL_SIZES[lvl] // NW
            s = STRIDES[lvl]
            inv_s = 1.0 / s
            lo = RANGE_LO[lvl]
            hi = RANGE_HI[lvl]
            base = wid * C

            def init_body(i, _):
                sl = pl.ds(i * 16, 16)
                ba_v[sl] = jnp.full((16,), INF, jnp.float32)
                bl_v[sl] = jnp.zeros((16,), jnp.float32)
                br_v[sl] = jnp.zeros((16,), jnp.float32)
                bc_v[sl] = jnp.zeros((16,), jnp.float32)
                return 0

            lax.fori_loop(0, (C + 32) // 16, init_body, 0, unroll=False)

            def ann_group_body(g, _, b=b, base=base, C=C, s=s, inv_s=inv_s,
                               lo=lo, hi=hi):
                gs = pl.ds(g * 16, 16)
                lvec = ann_v[b, 0, gs]
                k0v = (lvec * inv_s).astype(jnp.int32)
                grp_rel = (k0v[15] + 5 >= base) & (k0v[0] < base + C)

                @pl.when(grp_rel)
                def _scan_group():
                    rvec = ann_v[b, 1, gs]
                    cvec = ann_v[b, 2, gs]
                    for j in range(16):
                        k0 = k0v[j]

                        @pl.when((k0 + 5 >= base) & (k0 < base + C))
                        def _process(j=j, k0=k0):
                            l = lvec[j]
                            r = rvec[j]
                            c = cvec[j]
                            kvec = k0 + lane
                            af = kvec.astype(jnp.float32) * s
                            zero_v = af * 0.0
                            l_v = l + zero_v
                            r_v = r + zero_v
                            c_v = c + zero_v
                            rad_s = (DOWNBEAT_RADIUS
                                     - (DOWNBEAT_RADIUS - BEAT_RADIUS) * c) * s
                            in_box = (af >= l_v) & (af <= jnp.minimum(r_v, l + rad_s + zero_v))
                            l_star = af - l_v
                            r_star = r_v - af
                            mx = jnp.maximum(l_star, r_star)
                            valid = (in_box & (mx >= lo) & (mx <= hi)
                                     & (kvec >= base) & (kvec < base + C)
                                     & (lane < 6))
                            area_v = r_v - l_v
                            off = k0 - base + _PAD
                            sl = pl.ds(off, 16)
                            ba = ba_v[sl]
                            better = valid & (area_v < ba)
                            ba_v[sl] = jnp.where(better, area_v, ba)
                            bl_v[sl] = jnp.where(better, l_v, bl_v[sl])
                            br_v[sl] = jnp.where(better, r_v, br_v[sl])
                            bc_v[sl] = jnp.where(better, c_v, bc_v[sl])

                return 0

            lax.fori_loop(0, M // 16, ann_group_body, 0, unroll=False)

            def epi_body(i, _, base=base, s=s, inv_s=inv_s):
                sl = pl.ds(_PAD + i * 16, 16)
                osl = pl.ds(i * 16, 16)
                kvec = base + i * 16 + lane
                af = kvec.astype(jnp.float32) * s
                ba = ba_v[sl]
                posf = jnp.sign(INF - ba)
                bl = bl_v[sl]
                br = br_v[sl]
                bc = bc_v[sl]
                st_v[0, osl] = posf
                st_v[1, osl] = bc * posf
                st_v[2, osl] = (af - bl) * inv_s
                st_v[3, osl] = (br - af) * inv_s
                return 0

            lax.fori_loop(0, C // 16, epi_body, 0, unroll=False)

            gstart = LEVEL_OFFS[lvl] + base
            for f in range(4):
                pltpu.sync_copy(st_v.at[f, pl.ds(0, C)],
                                out_hbm.at[b, f, pl.ds(gstart, C)])


def _make_sc_assign():
    mesh = plsc.VectorSubcoreMesh(core_axis_name="c", subcore_axis_name="s")
    return pl.kernel(
        _sc_assign_body,
        out_type=jax.ShapeDtypeStruct((B, 4, N_TOTAL), jnp.float32),
        mesh=mesh,
        scratch_types=[
            pltpu.VMEM((B, 3, M), jnp.float32),
            pltpu.VMEM((CMAX + 32,), jnp.float32),
            pltpu.VMEM((CMAX + 32,), jnp.float32),
            pltpu.VMEM((CMAX + 32,), jnp.float32),
            pltpu.VMEM((CMAX + 32,), jnp.float32),
            pltpu.VMEM((4, CMAX), jnp.float32),
        ],
    )


BLK = 4096
NBLK = N_TOTAL // BLK


def _loss_kernel(cls_ref, reg_ref, lef_ref, asg_ref, out_ref):
    j = pl.program_id(1)

    posf = asg_ref[0, 0]
    acls = asg_ref[0, 1]
    nl = asg_ref[0, 2]
    nr = asg_ref[0, 3]

    t1 = posf * (acls > 0.5).astype(jnp.float32)
    t0 = posf - t1
    p = jnp.clip(cls_ref[0], EPS, 1.0 - EPS)
    p0 = p[0]
    p1 = p[1]

    def _focal(t, q):
        pos_term = 0.25 * (1.0 - q) * (1.0 - q) * (-jnp.log(q))
        neg_term = 0.75 * q * q * (-jnp.log(1.0 - q))
        return jnp.where(t == 1.0, pos_term, neg_term)

    cls_sum = jnp.sum(_focal(t0, p0) + _focal(t1, p1))

    tl = jnp.maximum(nl, 1e-3)
    tr = jnp.maximum(nr, 1e-3)
    reg = reg_ref[0]
    pl_ = reg[0]
    pr_ = reg[1]
    inter = jnp.minimum(pl_, tl) + jnp.minimum(pr_, tr)
    union = jnp.maximum(pl_, tl) + jnp.maximum(pr_, tr)
    iou = jnp.clip(inter / (union + EPS), EPS, 1.0)
    reg_sum = jnp.sum(-jnp.log(iou) * posf)

    lt = jnp.clip(tr / (tl + tr + EPS), EPS, 1.0 - EPS)
    lp = jnp.clip(lef_ref[0, 0], EPS, 1.0 - EPS)
    lbce = -(lt * jnp.log(lp) + (1.0 - lt) * jnp.log(1.0 - lp))
    left_sum = jnp.sum(lbce * posf)

    pos_sum = jnp.sum(posf)

    lane = lax.broadcasted_iota(jnp.int32, (1, 1, 128), 2)
    acc = jnp.where(lane == 0, cls_sum,
                    jnp.where(lane == 1, reg_sum,
                              jnp.where(lane == 2, left_sum,
                                        jnp.where(lane == 3, pos_sum, 0.0))))

    @pl.when(j == 0)
    def _init():
        out_ref[...] = acc

    @pl.when(j > 0)
    def _acc():
        out_ref[...] += acc


def kernel(classifications, regressions, leftnesses, annotations):
    cls_t = classifications.transpose(0, 2, 1)
    reg_t = regressions.transpose(0, 2, 1)
    lef_t = leftnesses.transpose(0, 2, 1)
    ann_t = annotations.transpose(0, 2, 1)

    assign = _make_sc_assign()(ann_t)

    partials = pl.pallas_call(
        _loss_kernel,
        grid=(B, NBLK),
        in_specs=[
            pl.BlockSpec((1, 2, BLK), lambda b, j: (b, 0, j)),
            pl.BlockSpec((1, 2, BLK), lambda b, j: (b, 0, j)),
            pl.BlockSpec((1, 1, BLK), lambda b, j: (b, 0, j)),
            pl.BlockSpec((1, 4, BLK), lambda b, j: (b, 0, j)),
        ],
        out_specs=pl.BlockSpec((1, 1, 128), lambda b, j: (b, 0, 0)),
        out_shape=jax.ShapeDtypeStruct((B, 1, 128), jnp.float32),
    )(cls_t, reg_t, lef_t, assign)

    sums = partials[:, 0, :4]
    num_pos = jnp.maximum(sums[:, 3], 1.0)
    per_batch = (sums[:, 0] + sums[:, 1] + sums[:, 2]) / num_pos
    return jnp.sum(per_batch) / float(B)


# speedup vs baseline: 23.1848x; 1.1115x over previous
import functools

import jax
import jax.numpy as jnp
from jax import lax
from jax.experimental import pallas as pl
from jax.experimental.pallas import tpu as pltpu
from jax.experimental.pallas import tpu_sc as plsc

INF = 1e8
LEVEL_SIZES = (32768, 16384, 8192)
LEVEL_OFFS = (0, 32768, 49152)
STRIDES = (1.0, 2.0, 4.0)
B = 2
M = 256
N_TOTAL = 57344
AUDIO_TARGET_RATE = 22050.0 / 256.0
BEAT_RADIUS = 2.5
DOWNBEAT_RADIUS = 4.5
EPS = 1e-6

_EDGE0 = 0.35 + (0.7 - 0.35) / 2.0
_EDGE1 = 0.7 + (1.4 - 0.7) / 2.0
RANGE_LO = (-1.0 * AUDIO_TARGET_RATE, _EDGE0 * AUDIO_TARGET_RATE, _EDGE1 * AUDIO_TARGET_RATE)
RANGE_HI = (_EDGE0 * AUDIO_TARGET_RATE, _EDGE1 * AUDIO_TARGET_RATE, 1000.0 * AUDIO_TARGET_RATE)

NW = 32
CMAX = LEVEL_SIZES[0] // NW
_PAD = 8


def _sc_assign_body(ann_hbm, out_hbm, ann_v, ba_v, bl_v, br_v, bc_v, st_v):
    wid = lax.axis_index("s") * 2 + lax.axis_index("c")
    pltpu.sync_copy(ann_hbm, ann_v)
    lane = lax.broadcasted_iota(jnp.int32, (16,), 0)

    for b in range(B):
        for lvl in range(3):
            C = LEVEL_SIZES[lvl] // NW
            s = STRIDES[lvl]
            inv_s = 1.0 / s
            lo = RANGE_LO[lvl]
            hi = RANGE_HI[lvl]
            base = wid * C

            def init_body(i, _):
                ba_v[pl.ds(i * 16, 16)] = jnp.full((16,), INF, jnp.float32)
                return 0

            lax.fori_loop(0, (C + 32) // 16, init_body, 0, unroll=4)

            def ann_group_body(g, _, b=b, base=base, C=C, s=s, inv_s=inv_s,
                               lo=lo, hi=hi):
                gs = pl.ds(g * 16, 16)
                lvec = ann_v[b, 0, gs]
                k0v = (lvec * inv_s).astype(jnp.int32)
                grp_rel = (k0v[15] + 5 >= base) & (k0v[0] < base + C)

                @pl.when(grp_rel)
                def _scan_group():
                    rvec = ann_v[b, 1, gs]
                    cvec = ann_v[b, 2, gs]
                    for j in range(16):
                        k0 = k0v[j]

                        @pl.when((k0 + 5 >= base) & (k0 < base + C))
                        def _process(j=j, k0=k0):
                            l = lvec[j]
                            r = rvec[j]
                            c = cvec[j]
                            kvec = k0 + lane
                            af = kvec.astype(jnp.float32) * s
                            zero_v = af * 0.0
                            l_v = l + zero_v
                            r_v = r + zero_v
                            c_v = c + zero_v
                            rad_s = (DOWNBEAT_RADIUS
                                     - (DOWNBEAT_RADIUS - BEAT_RADIUS) * c) * s
                            in_box = (af >= l_v) & (af <= jnp.minimum(r_v, l + rad_s + zero_v))
                            l_star = af - l_v
                            r_star = r_v - af
                            mx = jnp.maximum(l_star, r_star)
                            valid = (in_box & (mx >= lo) & (mx <= hi)
                                     & (kvec >= base) & (kvec < base + C)
                                     & (lane < 6))
                            area_v = r_v - l_v
                            off = k0 - base + _PAD
                            sl = pl.ds(off, 16)
                            ba = ba_v[sl]
                            better = valid & (area_v < ba)
                            ba_v[sl] = jnp.where(better, area_v, ba)
                            bl_v[sl] = jnp.where(better, l_v, bl_v[sl])
                            br_v[sl] = jnp.where(better, r_v, br_v[sl])
                            bc_v[sl] = jnp.where(better, c_v, bc_v[sl])

                return 0

            lax.fori_loop(0, M // 16, ann_group_body, 0, unroll=False)

            def epi_body(i, _, base=base, s=s, inv_s=inv_s):
                sl = pl.ds(_PAD + i * 16, 16)
                osl = pl.ds(i * 16, 16)
                kvec = base + i * 16 + lane
                af = kvec.astype(jnp.float32) * s
                ba = ba_v[sl]
                pos = ba < INF
                posf = jnp.sign(INF - ba)
                zero_v = posf * 0.0
                bl = jnp.where(pos, bl_v[sl], zero_v)
                br = jnp.where(pos, br_v[sl], zero_v)
                bc = jnp.where(pos, bc_v[sl], zero_v)
                st_v[0, osl] = posf
                st_v[1, osl] = bc
                st_v[2, osl] = (af - bl) * inv_s
                st_v[3, osl] = (br - af) * inv_s
                return 0

            lax.fori_loop(0, C // 16, epi_body, 0, unroll=2)

            gstart = LEVEL_OFFS[lvl] + base
            for f in range(4):
                pltpu.sync_copy(st_v.at[f, pl.ds(0, C)],
                                out_hbm.at[b, f, pl.ds(gstart, C)])


def _make_sc_assign():
    mesh = plsc.VectorSubcoreMesh(core_axis_name="c", subcore_axis_name="s")
    return pl.kernel(
        _sc_assign_body,
        out_type=jax.ShapeDtypeStruct((B, 4, N_TOTAL), jnp.float32),
        mesh=mesh,
        scratch_types=[
            pltpu.VMEM((B, 3, M), jnp.float32),
            pltpu.VMEM((CMAX + 32,), jnp.float32),
            pltpu.VMEM((CMAX + 32,), jnp.float32),
            pltpu.VMEM((CMAX + 32,), jnp.float32),
            pltpu.VMEM((CMAX + 32,), jnp.float32),
            pltpu.VMEM((4, CMAX), jnp.float32),
        ],
    )


BLK = 8192
NBLK = N_TOTAL // BLK


def _loss_kernel(cls_ref, reg_ref, lef_ref, asg_ref, out_ref):
    j = pl.program_id(1)

    posf = asg_ref[0, 0]
    acls = asg_ref[0, 1]
    nl = asg_ref[0, 2]
    nr = asg_ref[0, 3]

    t1 = posf * (acls > 0.5).astype(jnp.float32)
    t0 = posf - t1
    p = jnp.clip(cls_ref[0], EPS, 1.0 - EPS)
    p0 = p[0]
    p1 = p[1]

    def _focal(t, q):
        pos_term = 0.25 * (1.0 - q) * (1.0 - q) * (-jnp.log(q))
        neg_term = 0.75 * q * q * (-jnp.log(1.0 - q))
        return jnp.where(t == 1.0, pos_term, neg_term)

    cls_sum = jnp.sum(_focal(t0, p0) + _focal(t1, p1))

    tl = jnp.maximum(nl, 1e-3)
    tr = jnp.maximum(nr, 1e-3)
    reg = reg_ref[0]
    pl_ = reg[0]
    pr_ = reg[1]
    inter = jnp.minimum(pl_, tl) + jnp.minimum(pr_, tr)
    union = jnp.maximum(pl_, tl) + jnp.maximum(pr_, tr)
    iou = jnp.clip(inter / (union + EPS), EPS, 1.0)
    reg_sum = jnp.sum(-jnp.log(iou) * posf)

    lt = jnp.clip(tr / (tl + tr + EPS), EPS, 1.0 - EPS)
    lp = jnp.clip(lef_ref[0, 0], EPS, 1.0 - EPS)
    lbce = -(lt * jnp.log(lp) + (1.0 - lt) * jnp.log(1.0 - lp))
    left_sum = jnp.sum(lbce * posf)

    pos_sum = jnp.sum(posf)

    lane = lax.broadcasted_iota(jnp.int32, (1, 1, 128), 2)
    acc = jnp.where(lane == 0, cls_sum,
                    jnp.where(lane == 1, reg_sum,
                              jnp.where(lane == 2, left_sum,
                                        jnp.where(lane == 3, pos_sum, 0.0))))

    @pl.when(j == 0)
    def _init():
        out_ref[...] = acc

    @pl.when(j > 0)
    def _acc():
        out_ref[...] += acc


def kernel(classifications, regressions, leftnesses, annotations):
    cls_t = classifications.transpose(0, 2, 1)
    reg_t = regressions.transpose(0, 2, 1)
    lef_t = leftnesses.transpose(0, 2, 1)
    ann_t = annotations.transpose(0, 2, 1)

    assign = _make_sc_assign()(ann_t)

    partials = pl.pallas_call(
        _loss_kernel,
        grid=(B, NBLK),
        in_specs=[
            pl.BlockSpec((1, 2, BLK), lambda b, j: (b, 0, j)),
            pl.BlockSpec((1, 2, BLK), lambda b, j: (b, 0, j)),
            pl.BlockSpec((1, 1, BLK), lambda b, j: (b, 0, j)),
            pl.BlockSpec((1, 4, BLK), lambda b, j: (b, 0, j)),
        ],
        out_specs=pl.BlockSpec((1, 1, 128), lambda b, j: (b, 0, 0)),
        out_shape=jax.ShapeDtypeStruct((B, 1, 128), jnp.float32),
    )(cls_t, reg_t, lef_t, assign)

    sums = partials[:, 0, :4]
    num_pos = jnp.maximum(sums[:, 3], 1.0)
    per_batch = (sums[:, 0] + sums[:, 1] + sums[:, 2]) / num_pos
    return jnp.sum(per_batch) / float(B)


# speedup vs baseline: 25.3547x; 1.0936x over previous
import functools

import jax
import jax.numpy as jnp
from jax import lax
from jax.experimental import pallas as pl
from jax.experimental.pallas import tpu as pltpu
from jax.experimental.pallas import tpu_sc as plsc

INF = 1e8
LEVEL_SIZES = (32768, 16384, 8192)
LEVEL_OFFS = (0, 32768, 49152)
STRIDES = (1.0, 2.0, 4.0)
B = 2
M = 256
N_TOTAL = 57344
AUDIO_TARGET_RATE = 22050.0 / 256.0
BEAT_RADIUS = 2.5
DOWNBEAT_RADIUS = 4.5
EPS = 1e-6

_EDGE0 = 0.35 + (0.7 - 0.35) / 2.0
_EDGE1 = 0.7 + (1.4 - 0.7) / 2.0
RANGE_LO = (-1.0 * AUDIO_TARGET_RATE, _EDGE0 * AUDIO_TARGET_RATE, _EDGE1 * AUDIO_TARGET_RATE)
RANGE_HI = (_EDGE0 * AUDIO_TARGET_RATE, _EDGE1 * AUDIO_TARGET_RATE, 1000.0 * AUDIO_TARGET_RATE)

NW = 32
CMAX = LEVEL_SIZES[0] // NW
_PAD = 8


def _sc_assign_body(ann_hbm, out_hbm, ann_v, ba_v, bl_v, br_v, bc_v, st_v, sem):
    wid = lax.axis_index("s") * 2 + lax.axis_index("c")
    pltpu.sync_copy(ann_hbm, ann_v)
    lane = lax.broadcasted_iota(jnp.int32, (16,), 0)
    out_dmas = []

    for b in range(B):
        for lvl in range(3):
            C = LEVEL_SIZES[lvl] // NW
            s = STRIDES[lvl]
            inv_s = 1.0 / s
            lo = RANGE_LO[lvl]
            hi = RANGE_HI[lvl]
            base = wid * C

            def init_body(i, _):
                ba_v[pl.ds(i * 16, 16)] = jnp.full((16,), INF, jnp.float32)
                return 0

            lax.fori_loop(0, (C + 32) // 16, init_body, 0, unroll=4)

            def ann_group_body(g, _, b=b, base=base, C=C, s=s, inv_s=inv_s,
                               lo=lo, hi=hi):
                gs = pl.ds(g * 16, 16)
                lvec = ann_v[b, 0, gs]
                k0v = (lvec * inv_s).astype(jnp.int32)
                grp_rel = (k0v[15] + 5 >= base) & (k0v[0] < base + C)

                @pl.when(grp_rel)
                def _scan_group():
                    rvec = ann_v[b, 1, gs]
                    cvec = ann_v[b, 2, gs]
                    for j in range(16):
                        k0 = k0v[j]

                        @pl.when((k0 + 5 >= base) & (k0 < base + C))
                        def _process(j=j, k0=k0):
                            l = lvec[j]
                            r = rvec[j]
                            c = cvec[j]
                            kvec = k0 + lane
                            af = kvec.astype(jnp.float32) * s
                            zero_v = af * 0.0
                            l_v = l + zero_v
                            r_v = r + zero_v
                            c_v = c + zero_v
                            rad_s = (DOWNBEAT_RADIUS
                                     - (DOWNBEAT_RADIUS - BEAT_RADIUS) * c) * s
                            in_box = (af >= l_v) & (af <= jnp.minimum(r_v, l + rad_s + zero_v))
                            l_star = af - l_v
                            r_star = r_v - af
                            mx = jnp.maximum(l_star, r_star)
                            valid = (in_box & (mx >= lo) & (mx <= hi)
                                     & (kvec >= base) & (kvec < base + C)
                                     & (lane < 6))
                            area_v = r_v - l_v
                            off = k0 - base + _PAD
                            sl = pl.ds(off, 16)
                            ba = ba_v[sl]
                            better = valid & (area_v < ba)
                            ba_v[sl] = jnp.where(better, area_v, ba)
                            bl_v[sl] = jnp.where(better, l_v, bl_v[sl])
                            br_v[sl] = jnp.where(better, r_v, br_v[sl])
                            bc_v[sl] = jnp.where(better, c_v, bc_v[sl])

                return 0

            lax.fori_loop(0, M // 16, ann_group_body, 0, unroll=False)

            def epi_body(i, _, b=b, lvl=lvl, base=base, s=s, inv_s=inv_s):
                sl = pl.ds(_PAD + i * 16, 16)
                osl = pl.ds(i * 16, 16)
                kvec = base + i * 16 + lane
                af = kvec.astype(jnp.float32) * s
                ba = ba_v[sl]
                pos = ba < INF
                posf = jnp.sign(INF - ba)
                zero_v = posf * 0.0
                bl = jnp.where(pos, bl_v[sl], zero_v)
                br = jnp.where(pos, br_v[sl], zero_v)
                bc = jnp.where(pos, bc_v[sl], zero_v)
                st_v[b, lvl, 0, osl] = posf
                st_v[b, lvl, 1, osl] = bc
                st_v[b, lvl, 2, osl] = (af - bl) * inv_s
                st_v[b, lvl, 3, osl] = (br - af) * inv_s
                return 0

            lax.fori_loop(0, C // 16, epi_body, 0, unroll=2)

            gstart = LEVEL_OFFS[lvl] + base
            for f in range(4):
                out_dmas.append(pltpu.async_copy(
                    st_v.at[b, lvl, f, pl.ds(0, C)],
                    out_hbm.at[b, f, pl.ds(gstart, C)], sem))

    for dma in out_dmas:
        dma.wait()


def _make_sc_assign():
    mesh = plsc.VectorSubcoreMesh(core_axis_name="c", subcore_axis_name="s")
    return pl.kernel(
        _sc_assign_body,
        out_type=jax.ShapeDtypeStruct((B, 4, N_TOTAL), jnp.float32),
        mesh=mesh,
        scratch_types=[
            pltpu.VMEM((B, 3, M), jnp.float32),
            pltpu.VMEM((CMAX + 32,), jnp.float32),
            pltpu.VMEM((CMAX + 32,), jnp.float32),
            pltpu.VMEM((CMAX + 32,), jnp.float32),
            pltpu.VMEM((CMAX + 32,), jnp.float32),
            pltpu.VMEM((B, 3, 4, CMAX), jnp.float32),
            pltpu.SemaphoreType.DMA,
        ],
    )


BLK = 14336
NBLK = N_TOTAL // BLK


def _loss_kernel(cls_ref, reg_ref, lef_ref, asg_ref, out_ref):
    j = pl.program_id(1)

    posf = asg_ref[0, 0]
    acls = asg_ref[0, 1]
    nl = asg_ref[0, 2]
    nr = asg_ref[0, 3]

    t1 = posf * (acls > 0.5).astype(jnp.float32)
    t0 = posf - t1
    p = jnp.clip(cls_ref[0], EPS, 1.0 - EPS)
    p0 = p[0]
    p1 = p[1]

    def _focal(t, q):
        pos_term = 0.25 * (1.0 - q) * (1.0 - q) * (-jnp.log(q))
        neg_term = 0.75 * q * q * (-jnp.log(1.0 - q))
        return jnp.where(t == 1.0, pos_term, neg_term)

    cls_sum = jnp.sum(_focal(t0, p0) + _focal(t1, p1))

    tl = jnp.maximum(nl, 1e-3)
    tr = jnp.maximum(nr, 1e-3)
    reg = reg_ref[0]
    pl_ = reg[0]
    pr_ = reg[1]
    inter = jnp.minimum(pl_, tl) + jnp.minimum(pr_, tr)
    union = jnp.maximum(pl_, tl) + jnp.maximum(pr_, tr)
    iou = jnp.clip(inter / (union + EPS), EPS, 1.0)
    reg_sum = jnp.sum(-jnp.log(iou) * posf)

    lt = jnp.clip(tr / (tl + tr + EPS), EPS, 1.0 - EPS)
    lp = jnp.clip(lef_ref[0, 0], EPS, 1.0 - EPS)
    lbce = -(lt * jnp.log(lp) + (1.0 - lt) * jnp.log(1.0 - lp))
    left_sum = jnp.sum(lbce * posf)

    pos_sum = jnp.sum(posf)

    lane = lax.broadcasted_iota(jnp.int32, (1, 1, 128), 2)
    acc = jnp.where(lane == 0, cls_sum,
                    jnp.where(lane == 1, reg_sum,
                              jnp.where(lane == 2, left_sum,
                                        jnp.where(lane == 3, pos_sum, 0.0))))

    @pl.when(j == 0)
    def _init():
        out_ref[...] = acc

    @pl.when(j > 0)
    def _acc():
        out_ref[...] += acc


def kernel(classifications, regressions, leftnesses, annotations):
    cls_t = classifications.transpose(0, 2, 1)
    reg_t = regressions.transpose(0, 2, 1)
    lef_t = leftnesses.transpose(0, 2, 1)
    ann_t = annotations.transpose(0, 2, 1)

    assign = _make_sc_assign()(ann_t)

    partials = pl.pallas_call(
        _loss_kernel,
        grid=(B, NBLK),
        in_specs=[
            pl.BlockSpec((1, 2, BLK), lambda b, j: (b, 0, j)),
            pl.BlockSpec((1, 2, BLK), lambda b, j: (b, 0, j)),
            pl.BlockSpec((1, 1, BLK), lambda b, j: (b, 0, j)),
            pl.BlockSpec((1, 4, BLK), lambda b, j: (b, 0, j)),
        ],
        out_specs=pl.BlockSpec((1, 1, 128), lambda b, j: (b, 0, 0)),
        out_shape=jax.ShapeDtypeStruct((B, 1, 128), jnp.float32),
    )(cls_t, reg_t, lef_t, assign)

    sums = partials[:, 0, :4]
    num_pos = jnp.maximum(sums[:, 3], 1.0)
    per_batch = (sums[:, 0] + sums[:, 1] + sums[:, 2]) / num_pos
    return jnp.sum(per_batch) / float(B)


# speedup vs baseline: 26.0217x; 1.0263x over previous
import functools

import jax
import jax.numpy as jnp
from jax import lax
from jax.experimental import pallas as pl
from jax.experimental.pallas import tpu as pltpu
from jax.experimental.pallas import tpu_sc as plsc

INF = 1e8
LEVEL_SIZES = (32768, 16384, 8192)
LEVEL_OFFS = (0, 32768, 49152)
STRIDES = (1.0, 2.0, 4.0)
B = 2
M = 256
N_TOTAL = 57344
AUDIO_TARGET_RATE = 22050.0 / 256.0
BEAT_RADIUS = 2.5
DOWNBEAT_RADIUS = 4.5
EPS = 1e-6

_EDGE0 = 0.35 + (0.7 - 0.35) / 2.0
_EDGE1 = 0.7 + (1.4 - 0.7) / 2.0
RANGE_LO = (-1.0 * AUDIO_TARGET_RATE, _EDGE0 * AUDIO_TARGET_RATE, _EDGE1 * AUDIO_TARGET_RATE)
RANGE_HI = (_EDGE0 * AUDIO_TARGET_RATE, _EDGE1 * AUDIO_TARGET_RATE, 1000.0 * AUDIO_TARGET_RATE)

NW = 32
CMAX = LEVEL_SIZES[0] // NW
_PAD = 8


def _sc_assign_body(ann_hbm, aux_hbm, out_hbm, ann_v, aux_v, ba_v, bl_v, br_v,
                    bc_v, st_v, sem):
    wid = lax.axis_index("s") * 2 + lax.axis_index("c")
    pltpu.sync_copy(ann_hbm, ann_v)
    pltpu.sync_copy(aux_hbm, aux_v)
    lane = lax.broadcasted_iota(jnp.int32, (16,), 0)
    lanef = lane.astype(jnp.float32)
    big_g = jnp.full((16,), float(M), jnp.float32)
    neg_g = jnp.full((16,), -1.0, jnp.float32)
    out_dmas = []

    for b in range(B):
        for lvl in range(3):
            C = LEVEL_SIZES[lvl] // NW
            s = STRIDES[lvl]
            inv_s = 1.0 / s
            lo = RANGE_LO[lvl]
            hi = RANGE_HI[lvl]
            base = wid * C

            def init_body(i, _):
                ba_v[pl.ds(i * 16, 16)] = jnp.full((16,), INF, jnp.float32)
                return 0

            lax.fori_loop(0, (C + 32) // 16, init_body, 0, unroll=4)

            def ann_group_body(g, _, b=b, base=base, C=C, s=s, inv_s=inv_s,
                               lo=lo, hi=hi):
                gs = pl.ds(g * 16, 16)
                lvec = ann_v[b, 0, gs]
                k0v = (lvec * inv_s).astype(jnp.int32)
                rvec = ann_v[b, 1, gs]
                cvec = ann_v[b, 2, gs]
                for j in range(16):
                    k0 = k0v[j]

                    @pl.when((k0 + 5 >= base) & (k0 < base + C))
                    def _process(j=j, k0=k0):
                        l = lvec[j]
                        r = rvec[j]
                        c = cvec[j]
                        kvec = k0 + lane
                        af = kvec.astype(jnp.float32) * s
                        zero_v = af * 0.0
                        l_v = l + zero_v
                        r_v = r + zero_v
                        c_v = c + zero_v
                        rad_s = (DOWNBEAT_RADIUS
                                 - (DOWNBEAT_RADIUS - BEAT_RADIUS) * c) * s
                        in_box = (af >= l_v) & (af <= jnp.minimum(r_v, l + rad_s + zero_v))
                        l_star = af - l_v
                        r_star = r_v - af
                        mx = jnp.maximum(l_star, r_star)
                        valid = (in_box & (mx >= lo) & (mx <= hi)
                                 & (kvec >= base) & (kvec < base + C)
                                 & (lane < 6))
                        area_v = r_v - l_v
                        off = k0 - base + _PAD
                        sl = pl.ds(off, 16)
                        ba = ba_v[sl]
                        better = valid & (area_v < ba)
                        ba_v[sl] = jnp.where(better, area_v, ba)
                        bl_v[sl] = jnp.where(better, l_v, bl_v[sl])
                        br_v[sl] = jnp.where(better, r_v, br_v[sl])
                        bc_v[sl] = jnp.where(better, c_v, bc_v[sl])

                return 0

            hf = aux_v[b, 0] * inv_s
            tf = aux_v[b, 1] * inv_s
            basef = wid * float(C)
            lim_lo = basef - 5.0
            lim_hi = basef + float(C)
            g_lo = wid * 0 + M // 16
            g_hi = wid * 0 - 1
            for g in range(M // 16 - 1, -1, -1):
                g_lo = jnp.where((tf[g] >= lim_lo) & (hf[g] < lim_hi), g, g_lo)
            for g in range(M // 16):
                g_hi = jnp.where((tf[g] >= lim_lo) & (hf[g] < lim_hi), g, g_hi)
            lax.fori_loop(g_lo, g_hi + 1, ann_group_body, 0, unroll=False)

            def epi_body(i, _, b=b, lvl=lvl, base=base, s=s, inv_s=inv_s):
                sl = pl.ds(_PAD + i * 16, 16)
                osl = pl.ds(i * 16, 16)
                kvec = base + i * 16 + lane
                af = kvec.astype(jnp.float32) * s
                ba = ba_v[sl]
                pos = ba < INF
                posf = jnp.sign(INF - ba)
                zero_v = posf * 0.0
                bl = jnp.where(pos, bl_v[sl], zero_v)
                br = jnp.where(pos, br_v[sl], zero_v)
                bc = jnp.where(pos, bc_v[sl], zero_v)
                st_v[b, lvl, 0, osl] = posf
                st_v[b, lvl, 1, osl] = bc
                st_v[b, lvl, 2, osl] = (af - bl) * inv_s
                st_v[b, lvl, 3, osl] = (br - af) * inv_s
                return 0

            lax.fori_loop(0, C // 16, epi_body, 0, unroll=2)

            gstart = LEVEL_OFFS[lvl] + base
            for f in range(4):
                out_dmas.append(pltpu.async_copy(
                    st_v.at[b, lvl, f, pl.ds(0, C)],
                    out_hbm.at[b, f, pl.ds(gstart, C)], sem))

    for dma in out_dmas:
        dma.wait()


def _make_sc_assign():
    mesh = plsc.VectorSubcoreMesh(core_axis_name="c", subcore_axis_name="s")
    return pl.kernel(
        _sc_assign_body,
        out_type=jax.ShapeDtypeStruct((B, 4, N_TOTAL), jnp.float32),
        mesh=mesh,
        scratch_types=[
            pltpu.VMEM((B, 3, M), jnp.float32),
            pltpu.VMEM((B, 2, 16), jnp.float32),
            pltpu.VMEM((CMAX + 32,), jnp.float32),
            pltpu.VMEM((CMAX + 32,), jnp.float32),
            pltpu.VMEM((CMAX + 32,), jnp.float32),
            pltpu.VMEM((CMAX + 32,), jnp.float32),
            pltpu.VMEM((B, 3, 4, CMAX), jnp.float32),
            pltpu.SemaphoreType.DMA,
        ],
    )


BLK = 14336
NBLK = N_TOTAL // BLK


def _loss_kernel(cls_ref, reg_ref, lef_ref, asg_ref, out_ref, acc_ref):
    j = pl.program_id(1)

    posf = asg_ref[0, 0]
    acls = asg_ref[0, 1]
    nl = asg_ref[0, 2]
    nr = asg_ref[0, 3]

    t1 = posf * (acls > 0.5).astype(jnp.float32)
    t0 = posf - t1
    p = jnp.clip(cls_ref[0], EPS, 1.0 - EPS)
    p0 = p[0]
    p1 = p[1]

    def _focal(t, q):
        is_pos = t == 1.0
        arg = jnp.where(is_pos, q, 1.0 - q)
        coef = jnp.where(is_pos, 0.25 * (1.0 - q) * (1.0 - q), 0.75 * q * q)
        return coef * (-jnp.log(arg))

    cls_sum = jnp.sum(_focal(t0, p0) + _focal(t1, p1))

    tl = jnp.maximum(nl, 1e-3)
    tr = jnp.maximum(nr, 1e-3)
    reg = reg_ref[0]
    pl_ = reg[0]
    pr_ = reg[1]
    inter = jnp.minimum(pl_, tl) + jnp.minimum(pr_, tr)
    union = jnp.maximum(pl_, tl) + jnp.maximum(pr_, tr)
    iou = jnp.clip(inter / (union + EPS), EPS, 1.0)
    reg_sum = jnp.sum(-jnp.log(iou) * posf)

    lt = jnp.clip(tr / (tl + tr + EPS), EPS, 1.0 - EPS)
    lp = jnp.clip(lef_ref[0, 0], EPS, 1.0 - EPS)
    lbce = -(lt * jnp.log(lp) + (1.0 - lt) * jnp.log(1.0 - lp))
    left_sum = jnp.sum(lbce * posf)

    pos_sum = jnp.sum(posf)

    b = pl.program_id(0)
    lane = lax.broadcasted_iota(jnp.int32, (1, 1, 128), 2)
    acc = jnp.where(lane == 0, cls_sum + reg_sum + left_sum,
                    jnp.where(lane == 1, pos_sum, 0.0))

    @pl.when(j == 0)
    def _init():
        acc_ref[pl.ds(b, 1)] = acc

    @pl.when(j > 0)
    def _acc():
        acc_ref[pl.ds(b, 1)] += acc

    @pl.when((b == B - 1) & (j == NBLK - 1))
    def _finalize():
        total = 0.0
        for bb in range(B):
            row = acc_ref[bb]
            lane2 = lax.broadcasted_iota(jnp.int32, (1, 128), 1)
            lsum = jnp.sum(jnp.where(lane2 == 0, row, 0.0))
            npos = jnp.maximum(jnp.sum(jnp.where(lane2 == 1, row, 0.0)), 1.0)
            total = total + lsum / npos
        out_ref[...] = jnp.where(lane[0] == 0, total / float(B), 0.0)


def kernel(classifications, regressions, leftnesses, annotations):
    cls_t = classifications.transpose(0, 2, 1)
    reg_t = regressions.transpose(0, 2, 1)
    lef_t = leftnesses.transpose(0, 2, 1)
    ann_t = annotations.transpose(0, 2, 1)
    starts_t = ann_t[:, 0, :]
    aux = jnp.stack([starts_t[:, 0::16], starts_t[:, 15::16]], axis=1)

    assign = _make_sc_assign()(ann_t, aux)

    out = pl.pallas_call(
        _loss_kernel,
        grid=(B, NBLK),
        in_specs=[
            pl.BlockSpec((1, 2, BLK), lambda b, j: (b, 0, j)),
            pl.BlockSpec((1, 2, BLK), lambda b, j: (b, 0, j)),
            pl.BlockSpec((1, 1, BLK), lambda b, j: (b, 0, j)),
            pl.BlockSpec((1, 4, BLK), lambda b, j: (b, 0, j)),
        ],
        out_specs=pl.BlockSpec((1, 128), lambda b, j: (0, 0)),
        out_shape=jax.ShapeDtypeStruct((1, 128), jnp.float32),
        scratch_shapes=[pltpu.VMEM((B, 1, 128), jnp.float32)],
    )(cls_t, reg_t, lef_t, assign)

    return out[0, 0]
